# baseline (speedup 1.0000x reference)
import functools

import jax
import jax.numpy as jnp
from jax import lax
from jax.experimental import pallas as pl
from jax.experimental.pallas import tpu as pltpu
from jax.experimental.pallas import tpu_sc as plsc

NC = 2
NS = 16
NW = NC * NS
GL = 128
K = 8
CH = K * GL

BLKA = 1024
BLKB = 1024


def _eye(n):
    i = lax.broadcasted_iota(jnp.int32, (n, n), 0)
    j = lax.broadcasted_iota(jnp.int32, (n, n), 1)
    return (i == j).astype(jnp.float32)


def _mxu_t(x):
    return lax.dot_general(
        x, _eye(x.shape[0]), (((0,), (0,)), ((), ())),
        precision=lax.Precision.HIGHEST,
        preferred_element_type=jnp.float32,
    )


def _pack_body(x_ref, o_ref):
    x = x_ref[...]
    blk = x.shape[1] // 2
    o_ref[:, 0:64] = _mxu_t(x[:, 0:blk])
    o_ref[:, 64:128] = _mxu_t(x[:, blk:])


def _unpack_body(x_ref, o_ref):
    x = x_ref[...]
    blk = x.shape[0]
    for w in range(blk // 64):
        t = _mxu_t(x[64 * w:64 * (w + 1), :])
        o_ref[:, 128 * w:128 * w + 64] = t[0:64, :]
        o_ref[:, 128 * w + 64:128 * (w + 1)] = t[64:128, :]


@functools.lru_cache(maxsize=None)
def _pack(cols, blk):
    nblk = pl.cdiv(cols // 2, blk)
    return pl.pallas_call(
        _pack_body,
        grid=(nblk,),
        in_specs=[pl.BlockSpec((64, 2 * blk), lambda i: (0, i))],
        out_specs=pl.BlockSpec((blk, 128), lambda i: (i, 0)),
        out_shape=jax.ShapeDtypeStruct((nblk * blk, 128), jnp.float32),
    )


@functools.lru_cache(maxsize=None)
def _unpack(rows, blk):
    nblk = pl.cdiv(rows, blk)
    return pl.pallas_call(
        _unpack_body,
        grid=(nblk,),
        in_specs=[pl.BlockSpec((blk, 128), lambda i: (i, 0))],
        out_specs=pl.BlockSpec((64, 2 * blk), lambda i: (0, i)),
        out_shape=jax.ShapeDtypeStruct((64, 2 * rows), jnp.float32),
    )


@functools.lru_cache(maxsize=None)
def _gather(v, d, nch, perm=True):
    mesh = plsc.VectorSubcoreMesh(core_axis_name="c", subcore_axis_name="s")

    @functools.partial(
        pl.kernel,
        mesh=mesh,
        out_type=jax.ShapeDtypeStruct((NW, nch, CH, d), jnp.float32),
        scratch_types=[
            pltpu.VMEM((nch * K * GL,), jnp.int32),
            pltpu.VMEM((nch * K * GL,), jnp.int32),
            pltpu.VMEM((CH, d), jnp.float32),
            pltpu.SemaphoreType.DMA,
        ],
        compiler_params=pltpu.CompilerParams(
            use_tc_tiling_on_sc=False,
            needs_layout_passes=False,
        ),
    )
    def k(table_hbm, tok_hbm, out_hbm, idx_v, idxp_v, rows_v, sem):
        wid = lax.axis_index("s") * NC + lax.axis_index("c")
        pltpu.sync_copy(tok_hbm.at[wid], idx_v)

        i16 = lax.iota(jnp.int32, 16)
        patt = 64 * (i16 % 2) + i16 // 2

        def permrow(r, carry):
            base = r * GL
            for t in range(8):
                if perm:
                    x = plsc.load_gather(idx_v, [base + patt + 8 * t])
                else:
                    x = idx_v[pl.ds(base + 16 * t, 16)]
                idxp_v[pl.ds(base + 16 * t, 16)] = x
            return carry

        lax.fori_loop(0, nch * K, permrow, 0)

        def chunk(c, carry):
            cps = [
                pltpu.async_copy(
                    table_hbm.at[idxp_v.at[pl.ds((c * K + j) * GL, GL)]],
                    rows_v.at[pl.ds(j * GL, GL)],
                    sem,
                )
                for j in range(K)
            ]
            for cp in cps:
                cp.wait()
            pltpu.sync_copy(rows_v, out_hbm.at[wid, c])
            return carry

        lax.fori_loop(0, nch, chunk, 0)

    return k


def _kernel_fast(tokens, table):
    v, d = table.shape
    flat = tokens.reshape(-1).astype(jnp.int32)
    b = flat.shape[0]

    packed = _pack(v, BLKA)(table.T)
    v_lin = 2 * packed.shape[0]
    table_lin = packed.reshape(-1).reshape(v_lin, d)

    u = flat % (2 * BLKA)
    slots = (flat - u) + 2 * (u % BLKA) + u // BLKA

    blk = NW * CH
    nch = b // blk
    tok3 = slots.reshape(NW, nch * K * GL)

    rows = _gather(v_lin, d, nch)(table_lin, tok3)

    out_t = _unpack(b // 2, BLKB)(rows.reshape(-1).reshape(b // 2, 128))
    return out_t.T


def _kernel_simple(tokens, table):
    v, d = table.shape
    flat = tokens.reshape(-1).astype(jnp.int32)
    b = flat.shape[0]
    blk = NW * CH
    pad = (-b) % blk
    if pad:
        flat = jnp.concatenate([flat, jnp.zeros((pad,), jnp.int32)])
    nch = flat.shape[0] // blk
    tok3 = flat.reshape(NW, nch * K * GL)
    out = _gather(v, d, nch, perm=False)(table, tok3)
    out = out.reshape(-1, d)
    if pad:
        out = out[:b]
    return out


def kernel(tokens, table):
    v, d = table.shape
    b = tokens.size
    if d == 64 and v % 2 == 0 and b % (NW * CH) == 0 and b % (2 * BLKB) == 0:
        return _kernel_fast(tokens, table)
    return _kernel_simple(tokens, table)


# speedup vs baseline: 1.6472x; 1.6472x over previous
import functools

import jax
import jax.numpy as jnp
from jax import lax
from jax.experimental import pallas as pl
from jax.experimental.pallas import tpu as pltpu
from jax.experimental.pallas import tpu_sc as plsc

NC = 2
NS = 16
NW = NC * NS
GL = 128
K = 8
CH = K * GL

BLKA = 1024
BLKB = 2048


def _eye(n):
    i = lax.broadcasted_iota(jnp.int32, (n, n), 0)
    j = lax.broadcasted_iota(jnp.int32, (n, n), 1)
    return (i == j).astype(jnp.float32)


def _mxu_t(x):
    return lax.dot_general(
        x, _eye(x.shape[0]), (((0,), (0,)), ((), ())),
        precision=lax.Precision.HIGHEST,
        preferred_element_type=jnp.float32,
    )


REG = 1024


def _pack_body(x_ref, o_ref):
    x = x_ref[...]
    blk = x.shape[1] // 2
    xs = jnp.concatenate([x[:, 0:blk], x[:, blk:]], axis=0)
    o_ref[...] = xs.T


def _unpack_body(x_ref, o_ref):
    x = x_ref[...]
    blk = x.shape[0]
    w = x.T
    h = REG // 2
    for r in range(blk // h):
        c0 = r * REG
        o_ref[:, c0:c0 + h] = w[0:64, r * h:(r + 1) * h]
        o_ref[:, c0 + h:c0 + REG] = w[64:128, r * h:(r + 1) * h]


@functools.lru_cache(maxsize=None)
def _pack(cols, blk):
    nblk = pl.cdiv(cols // 2, blk)
    return pl.pallas_call(
        _pack_body,
        grid=(nblk,),
        in_specs=[pl.BlockSpec((64, 2 * blk), lambda i: (0, i))],
        out_specs=pl.BlockSpec((blk, 128), lambda i: (i, 0)),
        out_shape=jax.ShapeDtypeStruct((nblk * blk, 128), jnp.float32),
    )


@functools.lru_cache(maxsize=None)
def _unpack(rows, blk):
    nblk = pl.cdiv(rows, blk)
    return pl.pallas_call(
        _unpack_body,
        grid=(nblk,),
        in_specs=[pl.BlockSpec((blk, 128), lambda i: (i, 0))],
        out_specs=pl.BlockSpec((64, 2 * blk), lambda i: (0, i)),
        out_shape=jax.ShapeDtypeStruct((64, 2 * rows), jnp.float32),
    )


@functools.lru_cache(maxsize=None)
def _gather(v, d, nch, perm=True):
    mesh = plsc.VectorSubcoreMesh(core_axis_name="c", subcore_axis_name="s")

    @functools.partial(
        pl.kernel,
        mesh=mesh,
        out_type=jax.ShapeDtypeStruct((NW, nch, CH, d), jnp.float32),
        scratch_types=[
            pltpu.VMEM((nch * K * GL,), jnp.int32),
            pltpu.VMEM((nch * K * GL,), jnp.int32),
            pltpu.VMEM((CH, d), jnp.float32),
            pltpu.SemaphoreType.DMA,
        ],
        compiler_params=pltpu.CompilerParams(
            use_tc_tiling_on_sc=False,
            needs_layout_passes=False,
        ),
    )
    def k(table_hbm, tok_hbm, out_hbm, idx_v, idxp_v, rows_v, sem):
        wid = lax.axis_index("s") * NC + lax.axis_index("c")
        pltpu.sync_copy(tok_hbm.at[wid], idx_v)

        i16 = lax.iota(jnp.int32, 16)
        patt = (REG // 2) * (i16 % 2) + i16 // 2

        def permrow(rr, carry):
            for k in range(8):
                g = rr * 8 + k
                if perm:
                    region = g // (REG // 16)
                    t = g - region * (REG // 16)
                    x = plsc.load_gather(idx_v, [region * REG + 8 * t + patt])
                else:
                    x = idx_v[pl.ds(g * 16, 16)]
                idxp_v[pl.ds(g * 16, 16)] = x
            return carry

        lax.fori_loop(0, nch * K * GL // 128, permrow, 0)

        def chunk(c, carry):
            cps = [
                pltpu.async_copy(
                    table_hbm.at[idxp_v.at[pl.ds((c * K + j) * GL, GL)]],
                    rows_v.at[pl.ds(j * GL, GL)],
                    sem,
                )
                for j in range(K)
            ]
            for cp in cps:
                cp.wait()
            pltpu.sync_copy(rows_v, out_hbm.at[wid, c])
            return carry

        lax.fori_loop(0, nch, chunk, 0)

    return k


def _kernel_fast(tokens, table):
    v, d = table.shape
    flat = tokens.reshape(-1).astype(jnp.int32)
    b = flat.shape[0]

    packed = _pack(v, BLKA)(table.T)
    v_lin = 2 * packed.shape[0]
    table_lin = packed.reshape(-1).reshape(v_lin, d)

    u = flat % (2 * BLKA)
    slots = (flat - u) + 2 * (u % BLKA) + u // BLKA

    blk = NW * CH
    nch = b // blk
    tok3 = slots.reshape(NW, nch * K * GL)

    rows = _gather(v_lin, d, nch)(table_lin, tok3)

    out_t = _unpack(b // 2, BLKB)(rows.reshape(-1).reshape(b // 2, 128))
    return out_t.T


def _kernel_simple(tokens, table):
    v, d = table.shape
    flat = tokens.reshape(-1).astype(jnp.int32)
    b = flat.shape[0]
    blk = NW * CH
    pad = (-b) % blk
    if pad:
        flat = jnp.concatenate([flat, jnp.zeros((pad,), jnp.int32)])
    nch = flat.shape[0] // blk
    tok3 = flat.reshape(NW, nch * K * GL)
    out = _gather(v, d, nch, perm=False)(table, tok3)
    out = out.reshape(-1, d)
    if pad:
        out = out[:b]
    return out


def kernel(tokens, table):
    v, d = table.shape
    b = tokens.size
    if d == 64 and v % 2 == 0 and b % (NW * CH) == 0 and b % (2 * BLKB) == 0:
        return _kernel_fast(tokens, table)
    return _kernel_simple(tokens, table)


# speedup vs baseline: 2.0969x; 1.2730x over previous
import functools

import jax
import jax.numpy as jnp
from jax import lax
from jax.experimental import pallas as pl
from jax.experimental.pallas import tpu as pltpu
from jax.experimental.pallas import tpu_sc as plsc

NC = 2
NS = 16
NW = NC * NS
GL = 128
K = 8
CH = K * GL

BLKA = 2048
BLKB = 4096


def _eye(n):
    i = lax.broadcasted_iota(jnp.int32, (n, n), 0)
    j = lax.broadcasted_iota(jnp.int32, (n, n), 1)
    return (i == j).astype(jnp.float32)


def _mxu_t(x):
    return lax.dot_general(
        x, _eye(x.shape[0]), (((0,), (0,)), ((), ())),
        precision=lax.Precision.HIGHEST,
        preferred_element_type=jnp.float32,
    )


REG = 1024


def _pack_body(x_ref, o_ref):
    x = x_ref[...]
    blk = x.shape[1] // 2
    xs = jnp.concatenate([x[:, 0:blk], x[:, blk:]], axis=0)
    o_ref[...] = xs.T


def _unpack_body(x_ref, o_ref):
    x = x_ref[...]
    blk = x.shape[0]
    w = x.T
    h = REG // 2
    for r in range(blk // h):
        c0 = r * REG
        o_ref[:, c0:c0 + h] = w[0:64, r * h:(r + 1) * h]
        o_ref[:, c0 + h:c0 + REG] = w[64:128, r * h:(r + 1) * h]


@functools.lru_cache(maxsize=None)
def _pack(cols, blk):
    nblk = pl.cdiv(cols // 2, blk)
    return pl.pallas_call(
        _pack_body,
        grid=(nblk,),
        in_specs=[pl.BlockSpec((64, 2 * blk), lambda i: (0, i))],
        out_specs=pl.BlockSpec((blk, 128), lambda i: (i, 0)),
        out_shape=jax.ShapeDtypeStruct((nblk * blk, 128), jnp.float32),
    )


@functools.lru_cache(maxsize=None)
def _unpack(rows, blk):
    nblk = pl.cdiv(rows, blk)
    return pl.pallas_call(
        _unpack_body,
        grid=(nblk,),
        in_specs=[pl.BlockSpec((blk, 128), lambda i: (i, 0))],
        out_specs=pl.BlockSpec((64, 2 * blk), lambda i: (0, i)),
        out_shape=jax.ShapeDtypeStruct((64, 2 * rows), jnp.float32),
    )


@functools.lru_cache(maxsize=None)
def _gather(v, d, nch, perm=True):
    mesh = plsc.VectorSubcoreMesh(core_axis_name="c", subcore_axis_name="s")

    @functools.partial(
        pl.kernel,
        mesh=mesh,
        out_type=jax.ShapeDtypeStruct((NW, nch, CH, d), jnp.float32),
        scratch_types=[
            pltpu.VMEM((nch * K * GL,), jnp.int32),
            pltpu.VMEM((nch * K * GL,), jnp.int32),
            pltpu.VMEM((CH, d), jnp.float32),
            pltpu.SemaphoreType.DMA,
        ],
        compiler_params=pltpu.CompilerParams(
            use_tc_tiling_on_sc=False,
            needs_layout_passes=False,
        ),
    )
    def k(table_hbm, tok_hbm, out_hbm, idx_v, idxp_v, rows_v, sem):
        wid = lax.axis_index("s") * NC + lax.axis_index("c")
        pltpu.sync_copy(tok_hbm.at[wid], idx_v)

        i16 = lax.iota(jnp.int32, 16)
        patt = (REG // 2) * (i16 % 2) + i16 // 2

        def permrow(rr, carry):
            for k in range(8):
                g = rr * 8 + k
                if perm:
                    region = g // (REG // 16)
                    t = g - region * (REG // 16)
                    x = plsc.load_gather(idx_v, [region * REG + 8 * t + patt])
                else:
                    x = idx_v[pl.ds(g * 16, 16)]
                idxp_v[pl.ds(g * 16, 16)] = x
            return carry

        lax.fori_loop(0, nch * K * GL // 128, permrow, 0)

        def chunk(c, carry):
            cps = [
                pltpu.async_copy(
                    table_hbm.at[idxp_v.at[pl.ds((c * K + j) * GL, GL)]],
                    rows_v.at[pl.ds(j * GL, GL)],
                    sem,
                )
                for j in range(K)
            ]
            for cp in cps:
                cp.wait()
            pltpu.sync_copy(rows_v, out_hbm.at[wid, c])
            return carry

        lax.fori_loop(0, nch, chunk, 0)

    return k


def _kernel_fast(tokens, table):
    v, d = table.shape
    flat = tokens.reshape(-1).astype(jnp.int32)
    b = flat.shape[0]

    packed = _pack(v, BLKA)(table.T)
    v_lin = 2 * packed.shape[0]
    table_lin = packed.reshape(-1).reshape(v_lin, d)

    u = flat % (2 * BLKA)
    slots = (flat - u) + 2 * (u % BLKA) + u // BLKA

    blk = NW * CH
    nch = b // blk
    tok3 = slots.reshape(NW, nch * K * GL)

    rows = _gather(v_lin, d, nch)(table_lin, tok3)

    out_t = _unpack(b // 2, BLKB)(rows.reshape(-1).reshape(b // 2, 128))
    return out_t.T


def _kernel_simple(tokens, table):
    v, d = table.shape
    flat = tokens.reshape(-1).astype(jnp.int32)
    b = flat.shape[0]
    blk = NW * CH
    pad = (-b) % blk
    if pad:
        flat = jnp.concatenate([flat, jnp.zeros((pad,), jnp.int32)])
    nch = flat.shape[0] // blk
    tok3 = flat.reshape(NW, nch * K * GL)
    out = _gather(v, d, nch, perm=False)(table, tok3)
    out = out.reshape(-1, d)
    if pad:
        out = out[:b]
    return out


def kernel(tokens, table):
    v, d = table.shape
    b = tokens.size
    if d == 64 and v % 2 == 0 and b % (NW * CH) == 0 and b % (2 * BLKB) == 0:
        return _kernel_fast(tokens, table)
    return _kernel_simple(tokens, table)


# speedup vs baseline: 2.4835x; 1.1844x over previous
import functools

import jax
import jax.numpy as jnp
from jax import lax
from jax.experimental import pallas as pl
from jax.experimental.pallas import tpu as pltpu
from jax.experimental.pallas import tpu_sc as plsc

NC = 2
NS = 16
NW = NC * NS
GL = 128
K = 8
CH = K * GL

BLKA = 4096
BLKB = 8192


def _eye(n):
    i = lax.broadcasted_iota(jnp.int32, (n, n), 0)
    j = lax.broadcasted_iota(jnp.int32, (n, n), 1)
    return (i == j).astype(jnp.float32)


def _mxu_t(x):
    return lax.dot_general(
        x, _eye(x.shape[0]), (((0,), (0,)), ((), ())),
        precision=lax.Precision.HIGHEST,
        preferred_element_type=jnp.float32,
    )


REG = 1024


def _pack_body(x_ref, o_ref):
    x = x_ref[...]
    blk = x.shape[1] // 2
    xs = jnp.concatenate([x[:, 0:blk], x[:, blk:]], axis=0)
    o_ref[...] = xs.T


def _unpack_body(x_ref, o_ref):
    x = x_ref[...]
    blk = x.shape[0]
    w = x.T
    h = REG // 2
    for r in range(blk // h):
        c0 = r * REG
        o_ref[:, c0:c0 + h] = w[0:64, r * h:(r + 1) * h]
        o_ref[:, c0 + h:c0 + REG] = w[64:128, r * h:(r + 1) * h]


@functools.lru_cache(maxsize=None)
def _pack(cols, blk):
    nblk = pl.cdiv(cols // 2, blk)
    return pl.pallas_call(
        _pack_body,
        grid=(nblk,),
        in_specs=[pl.BlockSpec((64, 2 * blk), lambda i: (0, i))],
        out_specs=pl.BlockSpec((blk, 128), lambda i: (i, 0)),
        out_shape=jax.ShapeDtypeStruct((nblk * blk, 128), jnp.float32),
    )


@functools.lru_cache(maxsize=None)
def _unpack(rows, blk):
    nblk = pl.cdiv(rows, blk)
    return pl.pallas_call(
        _unpack_body,
        grid=(nblk,),
        in_specs=[pl.BlockSpec((blk, 128), lambda i: (i, 0))],
        out_specs=pl.BlockSpec((64, 2 * blk), lambda i: (0, i)),
        out_shape=jax.ShapeDtypeStruct((64, 2 * rows), jnp.float32),
    )


@functools.lru_cache(maxsize=None)
def _gather(v, d, nch, perm=True):
    mesh = plsc.VectorSubcoreMesh(core_axis_name="c", subcore_axis_name="s")

    @functools.partial(
        pl.kernel,
        mesh=mesh,
        out_type=jax.ShapeDtypeStruct((NW, nch, CH, d), jnp.float32),
        scratch_types=[
            pltpu.VMEM((nch * K * GL,), jnp.int32),
            pltpu.VMEM((nch * K * GL,), jnp.int32),
            pltpu.VMEM((CH, d), jnp.float32),
            pltpu.SemaphoreType.DMA,
        ],
        compiler_params=pltpu.CompilerParams(
            use_tc_tiling_on_sc=False,
            needs_layout_passes=False,
        ),
    )
    def k(table_hbm, tok_hbm, out_hbm, idx_v, idxp_v, rows_v, sem):
        wid = lax.axis_index("s") * NC + lax.axis_index("c")
        pltpu.sync_copy(tok_hbm.at[wid], idx_v)

        i16 = lax.iota(jnp.int32, 16)
        patt = (REG // 2) * (i16 % 2) + i16 // 2

        def permrow(rr, carry):
            for k in range(8):
                g = rr * 8 + k
                if perm:
                    region = g // (REG // 16)
                    t = g - region * (REG // 16)
                    x = plsc.load_gather(idx_v, [region * REG + 8 * t + patt])
                else:
                    x = idx_v[pl.ds(g * 16, 16)]
                idxp_v[pl.ds(g * 16, 16)] = x
            return carry

        lax.fori_loop(0, nch * K * GL // 128, permrow, 0)

        def chunk(c, carry):
            cps = [
                pltpu.async_copy(
                    table_hbm.at[idxp_v.at[pl.ds((c * K + j) * GL, GL)]],
                    rows_v.at[pl.ds(j * GL, GL)],
                    sem,
                )
                for j in range(K)
            ]
            for cp in cps:
                cp.wait()
            pltpu.sync_copy(rows_v, out_hbm.at[wid, c])
            return carry

        lax.fori_loop(0, nch, chunk, 0)

    return k


def _kernel_fast(tokens, table):
    v, d = table.shape
    flat = tokens.reshape(-1).astype(jnp.int32)
    b = flat.shape[0]

    packed = _pack(v, BLKA)(table.T)
    v_lin = 2 * packed.shape[0]
    table_lin = packed.reshape(-1).reshape(v_lin, d)

    u = flat % (2 * BLKA)
    slots = (flat - u) + 2 * (u % BLKA) + u // BLKA

    blk = NW * CH
    nch = b // blk
    tok3 = slots.reshape(NW, nch * K * GL)

    rows = _gather(v_lin, d, nch)(table_lin, tok3)

    out_t = _unpack(b // 2, BLKB)(rows.reshape(-1).reshape(b // 2, 128))
    return out_t.T


def _kernel_simple(tokens, table):
    v, d = table.shape
    flat = tokens.reshape(-1).astype(jnp.int32)
    b = flat.shape[0]
    blk = NW * CH
    pad = (-b) % blk
    if pad:
        flat = jnp.concatenate([flat, jnp.zeros((pad,), jnp.int32)])
    nch = flat.shape[0] // blk
    tok3 = flat.reshape(NW, nch * K * GL)
    out = _gather(v, d, nch, perm=False)(table, tok3)
    out = out.reshape(-1, d)
    if pad:
        out = out[:b]
    return out


def kernel(tokens, table):
    v, d = table.shape
    b = tokens.size
    if d == 64 and v % 2 == 0 and b % (NW * CH) == 0 and b % (2 * BLKB) == 0:
        return _kernel_fast(tokens, table)
    return _kernel_simple(tokens, table)


# speedup vs baseline: 2.6466x; 1.0657x over previous
import functools

import jax
import jax.numpy as jnp
from jax import lax
from jax.experimental import pallas as pl
from jax.experimental.pallas import tpu as pltpu
from jax.experimental.pallas import tpu_sc as plsc

NC = 2
NS = 16
NW = NC * NS
GL = 128
K = 8
CH = K * GL

BLKA = 8192
BLKB = 16384


def _eye(n):
    i = lax.broadcasted_iota(jnp.int32, (n, n), 0)
    j = lax.broadcasted_iota(jnp.int32, (n, n), 1)
    return (i == j).astype(jnp.float32)


def _mxu_t(x):
    return lax.dot_general(
        x, _eye(x.shape[0]), (((0,), (0,)), ((), ())),
        precision=lax.Precision.HIGHEST,
        preferred_element_type=jnp.float32,
    )


REG = 1024


def _pack_body(x_ref, o_ref):
    x = x_ref[...]
    blk = x.shape[1] // 2
    xs = jnp.concatenate([x[:, 0:blk], x[:, blk:]], axis=0)
    o_ref[...] = xs.T


def _unpack_body(x_ref, o_ref):
    x = x_ref[...]
    blk = x.shape[0]
    w = x.T
    h = REG // 2
    for r in range(blk // h):
        c0 = r * REG
        o_ref[:, c0:c0 + h] = w[0:64, r * h:(r + 1) * h]
        o_ref[:, c0 + h:c0 + REG] = w[64:128, r * h:(r + 1) * h]


@functools.lru_cache(maxsize=None)
def _pack(cols, blk):
    nblk = pl.cdiv(cols // 2, blk)
    return pl.pallas_call(
        _pack_body,
        grid=(nblk,),
        in_specs=[pl.BlockSpec((64, 2 * blk), lambda i: (0, i))],
        out_specs=pl.BlockSpec((blk, 128), lambda i: (i, 0)),
        out_shape=jax.ShapeDtypeStruct((nblk * blk, 128), jnp.float32),
    )


@functools.lru_cache(maxsize=None)
def _unpack(rows, blk):
    nblk = pl.cdiv(rows, blk)
    return pl.pallas_call(
        _unpack_body,
        grid=(nblk,),
        in_specs=[pl.BlockSpec((blk, 128), lambda i: (i, 0))],
        out_specs=pl.BlockSpec((64, 2 * blk), lambda i: (0, i)),
        out_shape=jax.ShapeDtypeStruct((64, 2 * rows), jnp.float32),
    )


@functools.lru_cache(maxsize=None)
def _gather(v, d, nch, perm=True):
    mesh = plsc.VectorSubcoreMesh(core_axis_name="c", subcore_axis_name="s")

    @functools.partial(
        pl.kernel,
        mesh=mesh,
        out_type=jax.ShapeDtypeStruct((NW, nch, CH, d), jnp.float32),
        scratch_types=[
            pltpu.VMEM((nch * K * GL,), jnp.int32),
            pltpu.VMEM((nch * K * GL,), jnp.int32),
            pltpu.VMEM((CH, d), jnp.float32),
            pltpu.SemaphoreType.DMA,
        ],
        compiler_params=pltpu.CompilerParams(
            use_tc_tiling_on_sc=False,
            needs_layout_passes=False,
        ),
    )
    def k(table_hbm, tok_hbm, out_hbm, idx_v, idxp_v, rows_v, sem):
        wid = lax.axis_index("s") * NC + lax.axis_index("c")
        pltpu.sync_copy(tok_hbm.at[wid], idx_v)

        i16 = lax.iota(jnp.int32, 16)
        patt = (REG // 2) * (i16 % 2) + i16 // 2

        def permrow(rr, carry):
            for k in range(8):
                g = rr * 8 + k
                if perm:
                    region = g // (REG // 16)
                    t = g - region * (REG // 16)
                    x = plsc.load_gather(idx_v, [region * REG + 8 * t + patt])
                else:
                    x = idx_v[pl.ds(g * 16, 16)]
                idxp_v[pl.ds(g * 16, 16)] = x
            return carry

        lax.fori_loop(0, nch * K * GL // 128, permrow, 0)

        def chunk(c, carry):
            cps = [
                pltpu.async_copy(
                    table_hbm.at[idxp_v.at[pl.ds((c * K + j) * GL, GL)]],
                    rows_v.at[pl.ds(j * GL, GL)],
                    sem,
                )
                for j in range(K)
            ]
            for cp in cps:
                cp.wait()
            pltpu.sync_copy(rows_v, out_hbm.at[wid, c])
            return carry

        lax.fori_loop(0, nch, chunk, 0)

    return k


def _kernel_fast(tokens, table):
    v, d = table.shape
    flat = tokens.reshape(-1).astype(jnp.int32)
    b = flat.shape[0]

    packed = _pack(v, BLKA)(table.T)
    v_lin = 2 * packed.shape[0]
    table_lin = packed.reshape(-1).reshape(v_lin, d)

    u = flat % (2 * BLKA)
    slots = (flat - u) + 2 * (u % BLKA) + u // BLKA

    blk = NW * CH
    nch = b // blk
    tok3 = slots.reshape(NW, nch * K * GL)

    rows = _gather(v_lin, d, nch)(table_lin, tok3)

    out_t = _unpack(b // 2, BLKB)(rows.reshape(-1).reshape(b // 2, 128))
    return out_t.T


def _kernel_simple(tokens, table):
    v, d = table.shape
    flat = tokens.reshape(-1).astype(jnp.int32)
    b = flat.shape[0]
    blk = NW * CH
    pad = (-b) % blk
    if pad:
        flat = jnp.concatenate([flat, jnp.zeros((pad,), jnp.int32)])
    nch = flat.shape[0] // blk
    tok3 = flat.reshape(NW, nch * K * GL)
    out = _gather(v, d, nch, perm=False)(table, tok3)
    out = out.reshape(-1, d)
    if pad:
        out = out[:b]
    return out


def kernel(tokens, table):
    v, d = table.shape
    b = tokens.size
    if d == 64 and v % 2 == 0 and b % (NW * CH) == 0 and b % (2 * BLKB) == 0:
        return _kernel_fast(tokens, table)
    return _kernel_simple(tokens, table)


# speedup vs baseline: 2.6748x; 1.0107x over previous
import functools

import jax
import jax.numpy as jnp
from jax import lax
from jax.experimental import pallas as pl
from jax.experimental.pallas import tpu as pltpu
from jax.experimental.pallas import tpu_sc as plsc

NC = 2
NS = 16
NW = NC * NS
GL = 128
K = 8
CH = K * GL

BLKA = 16384
BLKB = 16384


def _eye(n):
    i = lax.broadcasted_iota(jnp.int32, (n, n), 0)
    j = lax.broadcasted_iota(jnp.int32, (n, n), 1)
    return (i == j).astype(jnp.float32)


def _mxu_t(x):
    return lax.dot_general(
        x, _eye(x.shape[0]), (((0,), (0,)), ((), ())),
        precision=lax.Precision.HIGHEST,
        preferred_element_type=jnp.float32,
    )


REG = 1024


def _pack_body(x_ref, o_ref):
    x = x_ref[...]
    blk = x.shape[1] // 2
    xs = jnp.concatenate([x[:, 0:blk], x[:, blk:]], axis=0)
    o_ref[...] = xs.T


def _unpack_body(x_ref, o_ref):
    x = x_ref[...]
    blk = x.shape[0]
    w = x.T
    h = REG // 2
    for r in range(blk // h):
        c0 = r * REG
        o_ref[:, c0:c0 + h] = w[0:64, r * h:(r + 1) * h]
        o_ref[:, c0 + h:c0 + REG] = w[64:128, r * h:(r + 1) * h]


@functools.lru_cache(maxsize=None)
def _pack(cols, blk):
    nblk = pl.cdiv(cols // 2, blk)
    return pl.pallas_call(
        _pack_body,
        grid=(nblk,),
        in_specs=[pl.BlockSpec((64, 2 * blk), lambda i: (0, i))],
        out_specs=pl.BlockSpec((blk, 128), lambda i: (i, 0)),
        out_shape=jax.ShapeDtypeStruct((nblk * blk, 128), jnp.float32),
    )


@functools.lru_cache(maxsize=None)
def _unpack(rows, blk):
    nblk = pl.cdiv(rows, blk)
    return pl.pallas_call(
        _unpack_body,
        grid=(nblk,),
        in_specs=[pl.BlockSpec((blk, 128), lambda i: (i, 0))],
        out_specs=pl.BlockSpec((64, 2 * blk), lambda i: (0, i)),
        out_shape=jax.ShapeDtypeStruct((64, 2 * rows), jnp.float32),
    )


@functools.lru_cache(maxsize=None)
def _gather(v, d, nch, perm=True):
    mesh = plsc.VectorSubcoreMesh(core_axis_name="c", subcore_axis_name="s")

    @functools.partial(
        pl.kernel,
        mesh=mesh,
        out_type=jax.ShapeDtypeStruct((NW, nch, CH, d), jnp.float32),
        scratch_types=[
            pltpu.VMEM((nch * K * GL,), jnp.int32),
            pltpu.VMEM((nch * K * GL,), jnp.int32),
            pltpu.VMEM((CH, d), jnp.float32),
            pltpu.SemaphoreType.DMA,
        ],
        compiler_params=pltpu.CompilerParams(
            use_tc_tiling_on_sc=False,
            needs_layout_passes=False,
        ),
    )
    def k(table_hbm, tok_hbm, out_hbm, idx_v, idxp_v, rows_v, sem):
        wid = lax.axis_index("s") * NC + lax.axis_index("c")
        pltpu.sync_copy(tok_hbm.at[wid], idx_v)

        i16 = lax.iota(jnp.int32, 16)
        patt = (REG // 2) * (i16 % 2) + i16 // 2

        def permrow(rr, carry):
            for k in range(8):
                g = rr * 8 + k
                if perm:
                    region = g // (REG // 16)
                    t = g - region * (REG // 16)
                    x = plsc.load_gather(idx_v, [region * REG + 8 * t + patt])
                else:
                    x = idx_v[pl.ds(g * 16, 16)]
                idxp_v[pl.ds(g * 16, 16)] = x
            return carry

        lax.fori_loop(0, nch * K * GL // 128, permrow, 0)

        def chunk(c, carry):
            cps = [
                pltpu.async_copy(
                    table_hbm.at[idxp_v.at[pl.ds((c * K + j) * GL, GL)]],
                    rows_v.at[pl.ds(j * GL, GL)],
                    sem,
                )
                for j in range(K)
            ]
            for cp in cps:
                cp.wait()
            pltpu.sync_copy(rows_v, out_hbm.at[wid, c])
            return carry

        lax.fori_loop(0, nch, chunk, 0)

    return k


def _kernel_fast(tokens, table):
    v, d = table.shape
    flat = tokens.reshape(-1).astype(jnp.int32)
    b = flat.shape[0]

    packed = _pack(v, BLKA)(table.T)
    v_lin = 2 * packed.shape[0]
    table_lin = packed.reshape(-1).reshape(v_lin, d)

    u = flat % (2 * BLKA)
    slots = (flat - u) + 2 * (u % BLKA) + u // BLKA

    blk = NW * CH
    nch = b // blk
    tok3 = slots.reshape(NW, nch * K * GL)

    rows = _gather(v_lin, d, nch)(table_lin, tok3)

    out_t = _unpack(b // 2, BLKB)(rows.reshape(-1).reshape(b // 2, 128))
    return out_t.T


def _kernel_simple(tokens, table):
    v, d = table.shape
    flat = tokens.reshape(-1).astype(jnp.int32)
    b = flat.shape[0]
    blk = NW * CH
    pad = (-b) % blk
    if pad:
        flat = jnp.concatenate([flat, jnp.zeros((pad,), jnp.int32)])
    nch = flat.shape[0] // blk
    tok3 = flat.reshape(NW, nch * K * GL)
    out = _gather(v, d, nch, perm=False)(table, tok3)
    out = out.reshape(-1, d)
    if pad:
        out = out[:b]
    return out


def kernel(tokens, table):
    v, d = table.shape
    b = tokens.size
    if d == 64 and v % 2 == 0 and b % (NW * CH) == 0 and b % (2 * BLKB) == 0:
        return _kernel_fast(tokens, table)
    return _kernel_simple(tokens, table)


# speedup vs baseline: 2.7244x; 1.0186x over previous
import functools

import jax
import jax.numpy as jnp
from jax import lax
from jax.experimental import pallas as pl
from jax.experimental.pallas import tpu as pltpu
from jax.experimental.pallas import tpu_sc as plsc

NC = 2
NS = 16
NW = NC * NS
GL = 128
K = 4
CH = K * GL

BLKA = 16384
BLKB = 16384


def _eye(n):
    i = lax.broadcasted_iota(jnp.int32, (n, n), 0)
    j = lax.broadcasted_iota(jnp.int32, (n, n), 1)
    return (i == j).astype(jnp.float32)


def _mxu_t(x):
    return lax.dot_general(
        x, _eye(x.shape[0]), (((0,), (0,)), ((), ())),
        precision=lax.Precision.HIGHEST,
        preferred_element_type=jnp.float32,
    )


REG = 1024


def _pack_body(x_ref, o_ref):
    x = x_ref[...]
    blk = x.shape[1] // 2
    xs = jnp.concatenate([x[:, 0:blk], x[:, blk:]], axis=0)
    o_ref[...] = xs.T


def _unpack_body(x_ref, o_ref):
    x = x_ref[...]
    blk = x.shape[0]
    w = x.T
    h = REG // 2
    for r in range(blk // h):
        c0 = r * REG
        o_ref[:, c0:c0 + h] = w[0:64, r * h:(r + 1) * h]
        o_ref[:, c0 + h:c0 + REG] = w[64:128, r * h:(r + 1) * h]


@functools.lru_cache(maxsize=None)
def _pack(cols, blk):
    nblk = pl.cdiv(cols // 2, blk)
    return pl.pallas_call(
        _pack_body,
        grid=(nblk,),
        in_specs=[pl.BlockSpec((64, 2 * blk), lambda i: (0, i))],
        out_specs=pl.BlockSpec((blk, 128), lambda i: (i, 0)),
        out_shape=jax.ShapeDtypeStruct((nblk * blk, 128), jnp.float32),
    )


@functools.lru_cache(maxsize=None)
def _unpack(rows, blk):
    nblk = pl.cdiv(rows, blk)
    return pl.pallas_call(
        _unpack_body,
        grid=(nblk,),
        in_specs=[pl.BlockSpec((blk, 128), lambda i: (i, 0))],
        out_specs=pl.BlockSpec((64, 2 * blk), lambda i: (0, i)),
        out_shape=jax.ShapeDtypeStruct((64, 2 * rows), jnp.float32),
    )


@functools.lru_cache(maxsize=None)
def _gather(v, d, nch, perm=True):
    mesh = plsc.VectorSubcoreMesh(core_axis_name="c", subcore_axis_name="s")

    @functools.partial(
        pl.kernel,
        mesh=mesh,
        out_type=jax.ShapeDtypeStruct((NW, nch, CH, d), jnp.float32),
        scratch_types=[
            pltpu.VMEM((nch * K * GL,), jnp.int32),
            pltpu.VMEM((nch * K * GL,), jnp.int32),
            pltpu.VMEM((2, CH, d), jnp.float32),
            pltpu.SemaphoreType.DMA,
            pltpu.SemaphoreType.DMA,
        ],
        compiler_params=pltpu.CompilerParams(
            use_tc_tiling_on_sc=False,
            needs_layout_passes=False,
        ),
    )
    def k(table_hbm, tok_hbm, out_hbm, idx_v, idxp_v, rows_v, gsem, wsem):
        wid = lax.axis_index("s") * NC + lax.axis_index("c")
        pltpu.sync_copy(tok_hbm.at[wid], idx_v)

        i16 = lax.iota(jnp.int32, 16)
        patt = (REG // 2) * (i16 % 2) + i16 // 2

        def permrow(rr, carry):
            for k in range(8):
                g = rr * 8 + k
                if perm:
                    region = g // (REG // 16)
                    t = g - region * (REG // 16)
                    x = plsc.load_gather(idx_v, [region * REG + 8 * t + patt])
                else:
                    x = idx_v[pl.ds(g * 16, 16)]
                idxp_v[pl.ds(g * 16, 16)] = x
            return carry

        lax.fori_loop(0, nch * K * GL // 128, permrow, 0)

        def chunk(c, carry):
            b = c % 2

            @pl.when(c >= 2)
            def _():
                pltpu.make_async_copy(
                    rows_v.at[b], out_hbm.at[wid, c - 2], wsem
                ).wait()

            cps = [
                pltpu.async_copy(
                    table_hbm.at[idxp_v.at[pl.ds((c * K + j) * GL, GL)]],
                    rows_v.at[b, pl.ds(j * GL, GL)],
                    gsem,
                )
                for j in range(K)
            ]
            for cp in cps:
                cp.wait()
            pltpu.async_copy(rows_v.at[b], out_hbm.at[wid, c], wsem)
            return carry

        lax.fori_loop(0, nch, chunk, 0)

        for b in range(2):
            pltpu.make_async_copy(
                rows_v.at[b], out_hbm.at[wid, nch - 2 + b], wsem
            ).wait()

    return k


def _kernel_fast(tokens, table):
    v, d = table.shape
    flat = tokens.reshape(-1).astype(jnp.int32)
    b = flat.shape[0]

    packed = _pack(v, BLKA)(table.T)
    v_lin = 2 * packed.shape[0]
    table_lin = packed.reshape(-1).reshape(v_lin, d)

    u = flat % (2 * BLKA)
    slots = (flat - u) + 2 * (u % BLKA) + u // BLKA

    blk = NW * CH
    nch = b // blk
    tok3 = slots.reshape(NW, nch * K * GL)

    rows = _gather(v_lin, d, nch)(table_lin, tok3)

    out_t = _unpack(b // 2, BLKB)(rows.reshape(-1).reshape(b // 2, 128))
    return out_t.T


def _kernel_simple(tokens, table):
    v, d = table.shape
    flat = tokens.reshape(-1).astype(jnp.int32)
    b = flat.shape[0]
    blk = NW * CH
    pad = (-b) % blk
    if pad:
        flat = jnp.concatenate([flat, jnp.zeros((pad,), jnp.int32)])
    nch = flat.shape[0] // blk
    tok3 = flat.reshape(NW, nch * K * GL)
    out = _gather(v, d, nch, perm=False)(table, tok3)
    out = out.reshape(-1, d)
    if pad:
        out = out[:b]
    return out


def kernel(tokens, table):
    v, d = table.shape
    b = tokens.size
    if d == 64 and v % 2 == 0 and b % (NW * CH) == 0 and b % (2 * BLKB) == 0:
        return _kernel_fast(tokens, table)
    return _kernel_simple(tokens, table)
